# baseline (speedup 1.0000x reference)
import functools

import jax
import jax.numpy as jnp
from jax import lax
from jax.experimental import pallas as pl
from jax.experimental.pallas import tpu as pltpu
from jax.experimental.pallas import tpu_sc as plsc

_NUM_CORES = 2
_NUM_SUBCORES = 16
_NW = _NUM_CORES * _NUM_SUBCORES
_LANES = 16

_CHUNK = 128


def _make_sc_lookup(n_rows: int, emb: int):
    per_w = n_rows // _NW
    n_chunks = per_w // _CHUNK
    assert per_w * _NW == n_rows and n_chunks * _CHUNK == per_w

    mesh = plsc.VectorSubcoreMesh(core_axis_name="c", subcore_axis_name="s")

    @functools.partial(
        pl.kernel,
        out_type=jax.ShapeDtypeStruct((n_rows, emb), jnp.float32),
        mesh=mesh,
        compiler_params=pltpu.CompilerParams(use_tc_tiling_on_sc=False),
        scratch_types=[
            pltpu.VMEM((_CHUNK,), jnp.int32),
            pltpu.VMEM((_CHUNK,), jnp.float32),
            pltpu.VMEM((_CHUNK, emb), jnp.float32),
            pltpu.SemaphoreType.DMA,
        ],
    )
    def lookup(idx_hbm, val_hbm, table_hbm, out_hbm, idx_v, val_v, rows_v, sem):
        wid = lax.axis_index("s") * _NUM_CORES + lax.axis_index("c")
        base = wid * per_w

        def chunk_body(ch, carry):
            off = base + ch * _CHUNK
            pltpu.sync_copy(idx_hbm.at[pl.ds(off, _CHUNK)], idx_v)
            pltpu.sync_copy(val_hbm.at[pl.ds(off, _CHUNK)], val_v)
            pltpu.async_copy(table_hbm.at[idx_v], rows_v, sem).wait()

            def group_body(g, c2):
                vv = val_v[pl.ds(g * _LANES, _LANES)]
                for j in range(_LANES):
                    r = g * _LANES + j
                    rows_v[r, :] = rows_v[r, :] * vv[j]
                return c2

            lax.fori_loop(0, _CHUNK // _LANES, group_body, 0)
            pltpu.sync_copy(rows_v, out_hbm.at[pl.ds(off, _CHUNK)])
            return carry

        lax.fori_loop(0, n_chunks, chunk_body, 0)

    return lookup


def kernel(id, value, table):
    b, f = id.shape
    _, emb = table.shape
    n_rows = b * f
    idx = id.reshape(n_rows).astype(jnp.int32)
    val = value.reshape(n_rows)
    out = _make_sc_lookup(n_rows, emb)(idx, val, table)
    return out.reshape(b, f, emb)


# speedup vs baseline: 1.1839x; 1.1839x over previous
import functools

import jax
import jax.numpy as jnp
from jax import lax
from jax.experimental import pallas as pl
from jax.experimental.pallas import tpu as pltpu
from jax.experimental.pallas import tpu_sc as plsc

_NUM_CORES = 2
_NUM_SUBCORES = 16
_NW = _NUM_CORES * _NUM_SUBCORES
_LANES = 16

_CHUNK = 128
_NBUF = 8


def _make_sc_lookup(n_rows: int, emb: int):
    per_w = n_rows // _NW
    n_chunks = per_w // _CHUNK
    outer = n_chunks // _NBUF
    assert per_w * _NW == n_rows and outer * _NBUF * _CHUNK == per_w

    mesh = plsc.VectorSubcoreMesh(core_axis_name="c", subcore_axis_name="s")

    @functools.partial(
        pl.kernel,
        out_type=jax.ShapeDtypeStruct((n_rows, emb), jnp.float32),
        mesh=mesh,
        compiler_params=pltpu.CompilerParams(use_tc_tiling_on_sc=False),
        scratch_types=[
            pltpu.VMEM((n_chunks, _CHUNK), jnp.int32),
            pltpu.VMEM((n_chunks, _CHUNK), jnp.float32),
            pltpu.VMEM((_NBUF, _CHUNK, emb), jnp.float32),
            pltpu.VMEM((_NBUF, _CHUNK, emb), jnp.float32),
            pltpu.SemaphoreType.DMA((_NBUF,)),
            pltpu.SemaphoreType.DMA((_NBUF,)),
        ],
    )
    def lookup(idx_hbm, val_hbm, table_hbm, out_hbm, idx_all, val_all,
               gbuf, obuf, gsem, osem):
        wid = lax.axis_index("s") * _NUM_CORES + lax.axis_index("c")
        base = wid * per_w
        pltpu.sync_copy(idx_hbm.at[wid], idx_all)
        pltpu.sync_copy(val_hbm.at[wid], val_all)

        def gather_desc(ch, b):
            return pltpu.make_async_copy(
                table_hbm.at[idx_all.at[ch]], gbuf.at[b], gsem.at[b])

        def out_desc(ch, b):
            return pltpu.make_async_copy(
                obuf.at[b], out_hbm.at[pl.ds(base + ch * _CHUNK, _CHUNK)],
                osem.at[b])

        for b in range(_NBUF):
            gather_desc(b, b).start()

        def outer_body(ch2, carry):
            for b in range(_NBUF):
                ch = ch2 * _NBUF + b
                gather_desc(ch, b).wait()

                @pl.when(ch2 > 0)
                def _wait_prev_out():
                    out_desc(ch, b).wait()

                def group_body(g, c2):
                    vv = val_all[ch, pl.ds(g * _LANES, _LANES)]
                    for j in range(_LANES):
                        r = g * _LANES + j
                        obuf[b, r, :] = gbuf[b, r, :] * vv[j]
                    return c2

                lax.fori_loop(0, _CHUNK // _LANES, group_body, 0)
                out_desc(ch, b).start()

                @pl.when(ch2 < outer - 1)
                def _next_gather():
                    gather_desc(ch + _NBUF, b).start()
            return carry

        lax.fori_loop(0, outer, outer_body, 0)
        for b in range(_NBUF):
            out_desc(0, b).wait()

    return lookup


def kernel(id, value, table):
    b, f = id.shape
    _, emb = table.shape
    n_rows = b * f
    per_w = n_rows // _NW
    n_chunks = per_w // _CHUNK
    idx = id.reshape(_NW, n_chunks, _CHUNK).astype(jnp.int32)
    val = value.reshape(_NW, n_chunks, _CHUNK)
    out = _make_sc_lookup(n_rows, emb)(idx, val, table)
    return out.reshape(b, f, emb)


# speedup vs baseline: 1.8673x; 1.5772x over previous
import functools

import jax
import jax.numpy as jnp
from jax import lax
from jax.experimental import pallas as pl
from jax.experimental.pallas import tpu as pltpu
from jax.experimental.pallas import tpu_sc as plsc

_NUM_CORES = 2
_NUM_SUBCORES = 16
_NW = _NUM_CORES * _NUM_SUBCORES
_LANES = 16

_BT = 128
_NTILES = 4


def _make_sc_lookup(batch: int, f_dim: int, emb: int):
    assert batch == _NW * _NTILES * _BT and emb == 2 * 8

    mesh = plsc.VectorSubcoreMesh(core_axis_name="c", subcore_axis_name="s")

    @functools.partial(
        pl.kernel,
        out_type=jax.ShapeDtypeStruct((f_dim, 2, batch // _BT, 8, _BT),
                                      jnp.float32),
        mesh=mesh,
        compiler_params=pltpu.CompilerParams(use_tc_tiling_on_sc=False,
                                             needs_layout_passes=False),
        scratch_types=[
            pltpu.VMEM((f_dim, _NTILES * _BT), jnp.int32),
            pltpu.VMEM((f_dim, _NTILES * _BT), jnp.float32),
            pltpu.VMEM((_NTILES, _BT, emb), jnp.float32),
            pltpu.VMEM((_NTILES, 2, 8, _BT), jnp.float32),
            pltpu.SemaphoreType.DMA((_NTILES,)),
            pltpu.SemaphoreType.DMA((_NTILES,)),
        ],
    )
    def lookup(idt_hbm, valt_hbm, table_hbm, out_hbm, idx_loc, val_loc,
               rows, ob, gsem, osem):
        wid = lax.axis_index("s") * _NUM_CORES + lax.axis_index("c")
        bcol = wid * (_NTILES * _BT)
        pltpu.sync_copy(idt_hbm.at[:, pl.ds(bcol, _NTILES * _BT)], idx_loc)
        pltpu.sync_copy(valt_hbm.at[:, pl.ds(bcol, _NTILES * _BT)], val_loc)

        def gather_desc(f, t):
            return pltpu.make_async_copy(
                table_hbm.at[idx_loc.at[f, pl.ds(t * _BT, _BT)]],
                rows.at[t], gsem.at[t])

        def out_desc(f, eb, t):
            return pltpu.make_async_copy(
                ob.at[t, eb], out_hbm.at[f, eb, wid * _NTILES + t],
                osem.at[t])

        for t in range(_NTILES):
            gather_desc(0, t).start()

        row_idx = [lax.iota(jnp.int32, _LANES) + c * _LANES
                   for c in range(_BT // _LANES)]

        def f_body(f, carry):
            for t in range(_NTILES):
                gather_desc(f, t).wait()

                @pl.when(f > 0)
                def _wait_prev_out():
                    out_desc(f, 0, t).wait()
                    out_desc(f, 1, t).wait()

                val_vecs = [val_loc[f, pl.ds(t * _BT + c * _LANES, _LANES)]
                            for c in range(_BT // _LANES)]
                for eb in range(2):
                    for es in range(8):
                        col = jnp.full((_LANES,), eb * 8 + es, jnp.int32)
                        for c in range(_BT // _LANES):
                            v = plsc.load_gather(rows.at[t],
                                                 [row_idx[c], col])
                            ob[t, eb, es, pl.ds(c * _LANES, _LANES)] = (
                                v * val_vecs[c])
                out_desc(f, 0, t).start()
                out_desc(f, 1, t).start()

                @pl.when(f < f_dim - 1)
                def _next_gather():
                    gather_desc(f + 1, t).start()
            return carry

        lax.fori_loop(0, f_dim, f_body, 0)
        for t in range(_NTILES):
            out_desc(f_dim - 1, 0, t).wait()
            out_desc(f_dim - 1, 1, t).wait()

    return lookup


def kernel(id, value, table):
    b, f = id.shape
    _, emb = table.shape
    idt = id.T.astype(jnp.int32)
    valt = value.T
    x = _make_sc_lookup(b, f, emb)(idt, valt, table)
    return x.transpose(2, 4, 0, 1, 3).reshape(b, f, emb)


# speedup vs baseline: 1.9141x; 1.0251x over previous
import functools

import jax
import jax.numpy as jnp
from jax import lax
from jax.experimental import pallas as pl
from jax.experimental.pallas import tpu as pltpu
from jax.experimental.pallas import tpu_sc as plsc

_NUM_CORES = 2
_NUM_SUBCORES = 16
_NW = _NUM_CORES * _NUM_SUBCORES
_LANES = 16

_BT = 128
_NTILES = 4
_NBUF = 8


def _make_sc_lookup(batch: int, f_dim: int, emb: int):
    assert batch == _NW * _NTILES * _BT and emb == 2 * 8
    n_chunks = f_dim * _NTILES
    outer = n_chunks // _NBUF
    assert outer * _NBUF == n_chunks

    mesh = plsc.VectorSubcoreMesh(core_axis_name="c", subcore_axis_name="s")

    @functools.partial(
        pl.kernel,
        out_type=jax.ShapeDtypeStruct((f_dim, 2, batch // _BT, 8, _BT),
                                      jnp.float32),
        mesh=mesh,
        compiler_params=pltpu.CompilerParams(use_tc_tiling_on_sc=False,
                                             needs_layout_passes=False),
        scratch_types=[
            pltpu.VMEM((f_dim, _NTILES * _BT), jnp.int32),
            pltpu.VMEM((f_dim, _NTILES * _BT), jnp.float32),
            pltpu.VMEM((_NBUF, _BT, emb), jnp.float32),
            pltpu.VMEM((_NBUF, 2, 8, _BT), jnp.float32),
            pltpu.SemaphoreType.DMA((_NBUF,)),
            pltpu.SemaphoreType.DMA((_NBUF,)),
        ],
    )
    def lookup(idt_hbm, valt_hbm, table_hbm, out_hbm, idx_loc, val_loc,
               rows, ob, gsem, osem):
        wid = lax.axis_index("s") * _NUM_CORES + lax.axis_index("c")
        bcol = wid * (_NTILES * _BT)
        pltpu.sync_copy(idt_hbm.at[:, pl.ds(bcol, _NTILES * _BT)], idx_loc)
        pltpu.sync_copy(valt_hbm.at[:, pl.ds(bcol, _NTILES * _BT)], val_loc)

        def gather_desc(f, t, s):
            return pltpu.make_async_copy(
                table_hbm.at[idx_loc.at[f, pl.ds(t * _BT, _BT)]],
                rows.at[s], gsem.at[s])

        def out_desc(f, eb, t, s):
            return pltpu.make_async_copy(
                ob.at[s, eb], out_hbm.at[f, eb, wid * _NTILES + t],
                osem.at[s])

        for k in range(_NBUF):
            gather_desc(k // _NTILES, k % _NTILES, k).start()

        row_idx = [lax.iota(jnp.int32, _LANES) + c * _LANES
                   for c in range(_BT // _LANES)]

        def outer_body(g, carry):
            for k in range(_NBUF):
                f = g * (_NBUF // _NTILES) + k // _NTILES
                t = k % _NTILES
                gather_desc(f, t, k).wait()

                @pl.when(g > 0)
                def _wait_prev_out():
                    out_desc(f, 0, t, k).wait()
                    out_desc(f, 1, t, k).wait()

                val_vecs = [val_loc[f, pl.ds(t * _BT + c * _LANES, _LANES)]
                            for c in range(_BT // _LANES)]
                for eb in range(2):
                    for es in range(8):
                        col = jnp.full((_LANES,), eb * 8 + es, jnp.int32)
                        for c in range(_BT // _LANES):
                            v = plsc.load_gather(rows.at[k],
                                                 [row_idx[c], col])
                            ob[k, eb, es, pl.ds(c * _LANES, _LANES)] = (
                                v * val_vecs[c])
                out_desc(f, 0, t, k).start()
                out_desc(f, 1, t, k).start()

                @pl.when(g < outer - 1)
                def _next_gather():
                    gather_desc(f + _NBUF // _NTILES, t, k).start()
            return carry

        lax.fori_loop(0, outer, outer_body, 0)
        for k in range(_NBUF):
            out_desc(f_dim - 1, 0, k % _NTILES, k).wait()
            out_desc(f_dim - 1, 1, k % _NTILES, k).wait()

    return lookup


def kernel(id, value, table):
    b, f = id.shape
    _, emb = table.shape
    idt = id.T.astype(jnp.int32)
    valt = value.T
    x = _make_sc_lookup(b, f, emb)(idt, valt, table)
    return x.transpose(2, 4, 0, 1, 3).reshape(b, f, emb)
